# baseline (speedup 1.0000x reference)
import functools

import jax
import jax.numpy as jnp
from jax import lax
from jax.experimental import pallas as pl
from jax.experimental.pallas import tpu as pltpu
from jax.experimental.pallas import tpu_sc as plsc

_B, _N, _C = 16, 65536, 10
_BN = _B * _N
_MIN_ROI = (0.0, -51.2, -5.0)
_GRID = 0.4

_NC, _NS, _L = 2, 16, 16
_NW = _NC * _NS
_PW = _BN // _NW
_CH = 4096
_NCH = _PW // _CH
_UNROLL = 4


def _ceil_idx(v, min_v):
    q = (v - jnp.float32(min_v)) / jnp.float32(_GRID)
    t = q.astype(jnp.int32)
    tf = t.astype(jnp.float32)
    return jnp.where(q > tf, t + 1, t)


def _sc_body(in_hbm, feat_hbm, idx_hbm, in_v, out_v):
    wid = lax.axis_index("s") * _NC + lax.axis_index("c")
    bval = wid // (_NW // _B)
    b16 = jnp.full((_L,), bval, dtype=jnp.int32)
    iota16 = lax.iota(jnp.int32, _L)
    x_min, y_min, z_min = _MIN_ROI

    def do_chunk(ch, carry):
        base = wid * _PW + ch * _CH
        pltpu.sync_copy(in_hbm.at[pl.ds(base * _C, _CH * _C)], in_v)
        pltpu.sync_copy(in_v, feat_hbm.at[pl.ds(base * _C, _CH * _C)])

        def step(i, c):
            for j in range(_UNROLL):
                rows = (i * _UNROLL + j) * _L + iota16
                off = rows * _C
                x = plsc.load_gather(in_v, [off])
                y = plsc.load_gather(in_v, [off + 1])
                z = plsc.load_gather(in_v, [off + 2])
                out = rows * 4
                plsc.store_scatter(out_v, [out], b16)
                plsc.store_scatter(out_v, [out + 1], _ceil_idx(z, z_min))
                plsc.store_scatter(out_v, [out + 2], _ceil_idx(y, y_min))
                plsc.store_scatter(out_v, [out + 3], _ceil_idx(x, x_min))
            return c

        lax.fori_loop(0, _CH // (_L * _UNROLL), step, 0)
        pltpu.sync_copy(out_v, idx_hbm.at[pl.ds(base * 4, _CH * 4)])
        return carry

    lax.fori_loop(0, _NCH, do_chunk, 0)


_sc_call = pl.kernel(
    _sc_body,
    out_type=(
        jax.ShapeDtypeStruct((_BN * _C,), jnp.float32),
        jax.ShapeDtypeStruct((_BN * 4,), jnp.int32),
    ),
    mesh=plsc.VectorSubcoreMesh(core_axis_name="c", subcore_axis_name="s"),
    compiler_params=pltpu.CompilerParams(needs_layout_passes=False),
    scratch_types=[
        pltpu.VMEM((_CH * _C,), jnp.float32),
        pltpu.VMEM((_CH * 4,), jnp.int32),
    ],
)


def kernel(rdr_sparse_cube):
    flat = rdr_sparse_cube.reshape(_BN * _C)
    feat, idx = _sc_call(flat)
    return feat.reshape(_BN, _C), idx.reshape(_BN, 4)


# speedup vs baseline: 13.8236x; 13.8236x over previous
import functools

import jax
import jax.numpy as jnp
from jax import lax
from jax.experimental import pallas as pl
from jax.experimental.pallas import tpu as pltpu
from jax.experimental.pallas import tpu_sc as plsc

_B, _N, _C = 16, 65536, 10
_BN = _B * _N
_MIN_ROI = (0.0, -51.2, -5.0)
_GRID = 0.4

_NC, _NS, _L = 2, 16, 16
_NW = _NC * _NS
_NG = _BN // 128
_GW = _NG // _NW
_GC = 32
_NCH = _GW // _GC


def _ceil_idx(v, min_v):
    q = (v - jnp.float32(min_v)) / jnp.float32(_GRID)
    t = q.astype(jnp.int32)
    tf = t.astype(jnp.float32)
    return jnp.where(q > tf, t + 1, t)


def _sc_body(in_hbm, feat_hbm, idx_hbm, in_v, out_v):
    wid = lax.axis_index("s") * _NC + lax.axis_index("c")
    b = wid // 2
    g0 = wid * _GW
    b_hi = b // 8
    b_lo = b % 8
    x_min, y_min, z_min = _MIN_ROI

    def do_chunk(ch, carry):
        g = g0 + ch * _GC
        tc = (g % 512)
        for c in range(_C):
            pltpu.sync_copy(
                in_hbm.at[c, b_hi, pl.ds(tc, _GC), b_lo, :], in_v.at[c])
        for c in range(_C):
            pltpu.sync_copy(
                in_v.at[c], feat_hbm.at[c // 8, pl.ds(g, _GC), c % 8, :])

        vb = jnp.full((_L,), b, dtype=jnp.int32)

        def step(i, cy):
            for k in range(128 // _L):
                s = pl.ds(k * _L, _L)
                x = in_v[0, i, s]
                y = in_v[1, i, s]
                z = in_v[2, i, s]
                out_v[i, 0, s] = vb
                out_v[i, 1, s] = _ceil_idx(z, z_min)
                out_v[i, 2, s] = _ceil_idx(y, y_min)
                out_v[i, 3, s] = _ceil_idx(x, x_min)
            return cy

        lax.fori_loop(0, _GC, step, 0)
        pltpu.sync_copy(out_v, idx_hbm.at[pl.ds(g, _GC), :, :])
        return carry

    lax.fori_loop(0, _NCH, do_chunk, 0)


_sc_call = pl.kernel(
    _sc_body,
    out_type=(
        jax.ShapeDtypeStruct((2, _NG, 8, 128), jnp.float32),
        jax.ShapeDtypeStruct((_NG, 4, 128), jnp.int32),
    ),
    mesh=plsc.VectorSubcoreMesh(core_axis_name="c", subcore_axis_name="s"),
    compiler_params=pltpu.CompilerParams(needs_layout_passes=False),
    scratch_types=[
        pltpu.VMEM((_C, _GC, 128), jnp.float32),
        pltpu.VMEM((_GC, 4, 128), jnp.int32),
    ],
)


def kernel(rdr_sparse_cube):
    xv = (rdr_sparse_cube.transpose(2, 0, 1)
          .reshape(_C, 2, 8, 512, 128)
          .transpose(0, 1, 3, 2, 4))
    f4, i3 = _sc_call(xv)
    feat = (f4.transpose(0, 2, 1, 3).reshape(16, _BN).T)[:, :_C]
    idx = i3.transpose(0, 2, 1).reshape(_BN, 4)
    return feat, idx
